# Initial kernel scaffold; baseline (speedup 1.0000x reference)
import jax
import jax.numpy as jnp
from jax.experimental import pallas as pl


def kernel(x, edge_index, comment_ids, W1, b1, W2, b2):
    raise NotImplementedError("write your pallas kernel here")


# speedup vs baseline: 15.7659x; 15.7659x over previous
import functools

import jax
import jax.numpy as jnp
from jax import lax
from jax.experimental import pallas as pl
from jax.experimental.pallas import tpu as pltpu
from jax.experimental.pallas import tpu_sc as plsc

N = 10000
E = 160000
D = 256
NSEG = 512
HALF = 128
NPAD = 10240
RB = 1024
GRID = NPAD // RB
NC = 2
NS = 16
EW = 125
EB = 80
NBUF = 2
IDP = 4
EPW = E // (NC * NS)
ROWS_PER_TILE = NPAD // NS
DEGROWS = NPAD // HALF
MAXNORM = 1.0 - 4e-3
EPSN = 1e-15

@functools.lru_cache(maxsize=None)
def _build_sc_degree():
    mesh = plsc.VectorSubcoreMesh(core_axis_name="c", subcore_axis_name="s",
                                  num_cores=NC, num_subcores=NS)
    return functools.partial(
        pl.kernel,
        out_type=jax.ShapeDtypeStruct((NC, DEGROWS, HALF), jnp.float32),
        mesh=mesh,
        scratch_types=[
            pltpu.VMEM((EPW + 16,), jnp.int32),
            pltpu.VMEM((DEGROWS, HALF), jnp.float32),
            pltpu.VMEM((DEGROWS,), jnp.int32),
            pltpu.VMEM_SHARED((DEGROWS, HALF), jnp.float32),
        ],
        compiler_params=pltpu.CompilerParams(use_tc_tiling_on_sc=False, needs_layout_passes=False),
    )(_sc_degree_body)


def _sc_degree_body(dst_hbm, out_hbm, dstbuf, hist, rowidx, acc):
    c = lax.axis_index("c")
    s = lax.axis_index("s")
    wid = s * NC + c

    def _zrow(i, _):
        def _zcol(k, _):
            hist[i, pl.ds(k * 16, 16)] = jnp.zeros((16,), jnp.float32)
            return 0
        return lax.fori_loop(0, HALF // 16, _zcol, 0)

    lax.fori_loop(0, DEGROWS, _zrow, 0)

    def _ridx(k, _):
        rowidx[pl.ds(k * 16, 16)] = lax.iota(jnp.int32, 16) + k * 16
        return 0

    lax.fori_loop(0, DEGROWS // 16, _ridx, 0)

    pltpu.sync_copy(hist.at[pl.ds(s * 5, 5)], acc.at[pl.ds(s * 5, 5)])
    plsc.subcore_barrier()

    pltpu.sync_copy(dst_hbm.at[wid], dstbuf.at[pl.ds(0, EPW)])

    ones = jnp.ones((16,), jnp.float32)

    def _acc(i, _):
        d = dstbuf[pl.ds(i * 16, 16)]
        plsc.addupdate_scatter(hist, [d >> 7, d & 127], ones)
        return 0

    nfull = EPW // 16
    lax.fori_loop(0, nfull, _acc, 0)
    rem = EPW - nfull * 16
    if rem:
        d = dstbuf[pl.ds(nfull * 16, 16)]
        m = lax.iota(jnp.int32, 16) < rem
        plsc.addupdate_scatter(hist, [d >> 7, d & 127], ones, mask=m)

    pltpu.sync_copy(hist, acc.at[rowidx], add=True)
    plsc.subcore_barrier()
    pltpu.sync_copy(acc.at[pl.ds(s * 5, 5)], out_hbm.at[c, pl.ds(s * 5, 5)])


@functools.lru_cache(maxsize=None)
def _build_sc_scatter():
    mesh = plsc.VectorSubcoreMesh(core_axis_name="c", subcore_axis_name="s",
                                  num_cores=NC, num_subcores=NS)
    return functools.partial(
        pl.kernel,
        out_type=jax.ShapeDtypeStruct((NC, NPAD, HALF), jnp.float32),
        mesh=mesh,
        scratch_types=[
            pltpu.VMEM((IDP, 2, EW), jnp.int32),
            pltpu.VMEM((NBUF, EW, HALF), jnp.float32),
            pltpu.VMEM((64, HALF), jnp.float32),
            pltpu.VMEM_SHARED((NPAD, HALF), jnp.float32),
            [pltpu.SemaphoreType.DMA] * NBUF,
            [pltpu.SemaphoreType.DMA] * NBUF,
            [pltpu.SemaphoreType.DMA] * IDP,
        ],
        compiler_params=pltpu.CompilerParams(use_tc_tiling_on_sc=False, needs_layout_passes=False),
    )(_sc_scatter_body)


def _sc_scatter_body(hs_hbm, eidx_hbm, zero_hbm, out_hbm,
                     idxring, rowbuf, zbuf, acc, gsems, ssems, esems):
    c = lax.axis_index("c")
    s = lax.axis_index("s")

    pltpu.sync_copy(zero_hbm, zbuf)
    base = s * ROWS_PER_TILE
    for k in range(ROWS_PER_TILE // 64):
        pltpu.sync_copy(zbuf, acc.at[pl.ds(base + k * 64, 64)])

    hsrc = hs_hbm.at[c]
    echunk = eidx_hbm.at[s]

    def _load_idx(j, p):
        pltpu.async_copy(echunk.at[j], idxring.at[p], esems[p])

    def _wait_idx(p):
        pltpu.make_async_copy(echunk.at[0], idxring.at[p], esems[p]).wait()

    def _gather(j_p, b):
        pltpu.async_copy(hsrc.at[idxring.at[j_p, 0]], rowbuf.at[b], gsems[b])

    def _wait_gather(b):
        pltpu.make_async_copy(hsrc.at[idxring.at[0, 0]], rowbuf.at[b],
                              gsems[b]).wait()

    def _scatter(j_p, b):
        pltpu.async_copy(rowbuf.at[b], acc.at[idxring.at[j_p, 1]], ssems[b],
                         add=True)

    def _wait_scatter(b):
        pltpu.make_async_copy(rowbuf.at[b], acc.at[idxring.at[0, 1]],
                              ssems[b]).wait()

    plsc.subcore_barrier()

    for p in range(IDP):
        _load_idx(p, p)
    for b in range(NBUF):
        _wait_idx(b)
        _gather(b, b)

    def _round(gg, _):
        j0 = IDP * gg
        for h in range(2):
            for q in range(NBUF):
                p = h * NBUF + q
                _wait_gather(q)
                _scatter(p, q)
            for q in range(NBUF):
                p = h * NBUF + q
                _wait_scatter(q)
                _load_idx(j0 + p + IDP, p)
                pn = (p + NBUF) % IDP
                _wait_idx(pn)
                _gather(pn, q)
        return 0

    lax.fori_loop(0, (EB - IDP) // IDP, _round, 0)
    for q in range(NBUF):
        _wait_gather(q)
        _scatter(q, q)
    for q in range(NBUF):
        _wait_scatter(q)
        _wait_idx(NBUF + q)
        _gather(NBUF + q, q)
    for q in range(NBUF):
        _wait_gather(q)
        _scatter(NBUF + q, q)
    for q in range(NBUF):
        _wait_scatter(q)

    plsc.subcore_barrier()
    pltpu.sync_copy(acc.at[pl.ds(base, ROWS_PER_TILE)],
                    out_hbm.at[c, pl.ds(base, ROWS_PER_TILE)])


def _tc1_body(x_ref, w_ref, d0_ref, d1_ref, hs_ref, h_ref, dinv_ref):
    deg = d0_ref[...] + d1_ref[...] + 1.0
    dinv = lax.rsqrt(jnp.maximum(deg, 1.0))
    h = jnp.dot(x_ref[...], w_ref[...], preferred_element_type=jnp.float32)
    hs = h * dinv
    hs_ref[0, :, :] = hs[:, :HALF]
    hs_ref[1, :, :] = hs[:, HALF:]
    h_ref[...] = h
    dinv_ref[...] = dinv


def _tc2_body(s_ref, h1_ref, dinv_ref, b1_ref, w2_ref, hs2_ref, h2_ref):
    dinv = dinv_ref[...]
    sfull = jnp.concatenate([s_ref[0], s_ref[1]], axis=1)
    h2 = jnp.tanh(dinv * sfull + (dinv * dinv) * h1_ref[...] + b1_ref[...])
    hh = jnp.dot(h2, w2_ref[...], preferred_element_type=jnp.float32)
    hss = hh * dinv
    hs2_ref[0, :, :] = hss[:, :HALF]
    hs2_ref[1, :, :] = hss[:, HALF:]
    h2_ref[...] = hh


def _tc3_body(s_ref, h2_ref, dinv_ref, b2_ref, ids_ref, out_ref,
              accs_ref, accc_ref):
    i = pl.program_id(0)

    @pl.when(i == 0)
    def _init():
        accs_ref[...] = jnp.zeros_like(accs_ref)
        accc_ref[...] = jnp.zeros_like(accc_ref)

    dinv = dinv_ref[...]
    sfull = jnp.concatenate([s_ref[0], s_ref[1]], axis=1)
    h = jnp.tanh(dinv * sfull + (dinv * dinv) * h2_ref[...] + b2_ref[...])
    nrm = jnp.maximum(jnp.sqrt(jnp.sum(h * h, axis=1, keepdims=True)), EPSN)
    hp = jnp.where(nrm > MAXNORM, h * (MAXNORM / nrm), h)
    n2 = jnp.maximum(jnp.sqrt(jnp.sum(hp * hp, axis=1, keepdims=True)), EPSN)
    z = jnp.minimum(n2, 1.0 - 1e-7)
    atz = 0.5 * jnp.log((1.0 + z) / (1.0 - z))
    u = (atz / n2) * hp
    ids = ids_ref[...]
    segi = lax.broadcasted_iota(jnp.int32, (NSEG, RB), 0)
    oh = (segi == ids).astype(jnp.float32)
    accs_ref[...] += lax.dot_general(
        oh, u, (((1,), (0,)), ((), ())), preferred_element_type=jnp.float32)
    accc_ref[...] += lax.dot_general(
        oh, jnp.ones((RB, 8), jnp.float32), (((1,), (0,)), ((), ())),
        preferred_element_type=jnp.float32)

    @pl.when(i == GRID - 1)
    def _final():
        cnt = accc_ref[...][:, :1]
        mean = accs_ref[...] / jnp.maximum(cnt, 1.0)
        nm = jnp.maximum(jnp.sqrt(jnp.sum(mean * mean, axis=1, keepdims=True)), EPSN)
        em = (jnp.tanh(nm) / nm) * mean
        ne = jnp.maximum(jnp.sqrt(jnp.sum(em * em, axis=1, keepdims=True)), EPSN)
        out_ref[...] = jnp.where(ne > MAXNORM, em * (MAXNORM / ne), em)


_f32 = jnp.float32


def _tc1(xp, W1, d0, d1):
    return pl.pallas_call(
        _tc1_body,
        grid=(GRID,),
        in_specs=[
            pl.BlockSpec((RB, D), lambda i: (i, 0)),
            pl.BlockSpec((D, D), lambda i: (0, 0)),
            pl.BlockSpec((RB, 1), lambda i: (i, 0)),
            pl.BlockSpec((RB, 1), lambda i: (i, 0)),
        ],
        out_specs=[
            pl.BlockSpec((NC, RB, HALF), lambda i: (0, i, 0)),
            pl.BlockSpec((RB, D), lambda i: (i, 0)),
            pl.BlockSpec((RB, 1), lambda i: (i, 0)),
        ],
        out_shape=[
            jax.ShapeDtypeStruct((NC, NPAD, HALF), _f32),
            jax.ShapeDtypeStruct((NPAD, D), _f32),
            jax.ShapeDtypeStruct((NPAD, 1), _f32),
        ],
    )(xp, W1, d0, d1)


def _tc2(s1, h1, dinv, b1, W2):
    return pl.pallas_call(
        _tc2_body,
        grid=(GRID,),
        in_specs=[
            pl.BlockSpec((NC, RB, HALF), lambda i: (0, i, 0)),
            pl.BlockSpec((RB, D), lambda i: (i, 0)),
            pl.BlockSpec((RB, 1), lambda i: (i, 0)),
            pl.BlockSpec((1, D), lambda i: (0, 0)),
            pl.BlockSpec((D, D), lambda i: (0, 0)),
        ],
        out_specs=[
            pl.BlockSpec((NC, RB, HALF), lambda i: (0, i, 0)),
            pl.BlockSpec((RB, D), lambda i: (i, 0)),
        ],
        out_shape=[
            jax.ShapeDtypeStruct((NC, NPAD, HALF), _f32),
            jax.ShapeDtypeStruct((NPAD, D), _f32),
        ],
    )(s1, h1, dinv, b1, W2)


def _tc3(s2, h2, dinv, b2, ids_row):
    return pl.pallas_call(
        _tc3_body,
        grid=(GRID,),
        in_specs=[
            pl.BlockSpec((NC, RB, HALF), lambda i: (0, i, 0)),
            pl.BlockSpec((RB, D), lambda i: (i, 0)),
            pl.BlockSpec((RB, 1), lambda i: (i, 0)),
            pl.BlockSpec((1, D), lambda i: (0, 0)),
            pl.BlockSpec((1, RB), lambda i: (0, i)),
        ],
        out_specs=pl.BlockSpec((NSEG, D), lambda i: (0, 0)),
        out_shape=jax.ShapeDtypeStruct((NSEG, D), _f32),
        scratch_shapes=[
            pltpu.VMEM((NSEG, D), _f32),
            pltpu.VMEM((NSEG, 8), _f32),
        ],
    )(s2, h2, dinv, b2, ids_row)


def kernel(x, edge_index, comment_ids, W1, b1, W2, b2):
    xp = jnp.pad(x, ((0, NPAD - N), (0, 0)))
    src_sc = edge_index[0].reshape(NS, EB, EW)
    dst_sc = edge_index[1].reshape(NS, EB, EW)
    eidx = jnp.stack([src_sc, dst_sc], axis=2)
    dst_deg = edge_index[1].reshape(NC * NS, EPW)
    ids_row = jnp.pad(comment_ids.astype(jnp.int32), (0, NPAD - N),
                      constant_values=-1).reshape(1, NPAD)
    zeros_hbm = jnp.zeros((64, HALF), _f32)

    degout = _build_sc_degree()(dst_deg)
    d0 = degout[0].reshape(NPAD, 1)
    d1 = degout[1].reshape(NPAD, 1)

    hs1, h1, dinv = _tc1(xp, W1, d0, d1)
    s1 = _build_sc_scatter()(hs1, eidx, zeros_hbm)
    hs2, h2 = _tc2(s1, h1, dinv, b1.reshape(1, D), W2)
    s2 = _build_sc_scatter()(hs2, eidx, zeros_hbm)
    out = _tc3(s2, h2, dinv, b2.reshape(1, D), ids_row)
    return out.reshape(16, 32, D)


# speedup vs baseline: 15.8265x; 1.0038x over previous
import functools

import jax
import jax.numpy as jnp
from jax import lax
from jax.experimental import pallas as pl
from jax.experimental.pallas import tpu as pltpu
from jax.experimental.pallas import tpu_sc as plsc

N = 10000
E = 160000
D = 256
NSEG = 512
HALF = 128
NPAD = 10240
RB = 1024
GRID = NPAD // RB
NC = 2
NS = 16
EW = 100
EB = 100
NBUF = 3
IDP = 6
EPW = E // (NC * NS)
ROWS_PER_TILE = NPAD // NS
DEGROWS = NPAD // HALF
MAXNORM = 1.0 - 4e-3
EPSN = 1e-15

@functools.lru_cache(maxsize=None)
def _build_sc_degree():
    mesh = plsc.VectorSubcoreMesh(core_axis_name="c", subcore_axis_name="s",
                                  num_cores=NC, num_subcores=NS)
    return functools.partial(
        pl.kernel,
        out_type=jax.ShapeDtypeStruct((NC, DEGROWS, HALF), jnp.float32),
        mesh=mesh,
        scratch_types=[
            pltpu.VMEM((EPW + 16,), jnp.int32),
            pltpu.VMEM((DEGROWS, HALF), jnp.float32),
            pltpu.VMEM((DEGROWS,), jnp.int32),
            pltpu.VMEM_SHARED((DEGROWS, HALF), jnp.float32),
        ],
        compiler_params=pltpu.CompilerParams(use_tc_tiling_on_sc=False, needs_layout_passes=False),
    )(_sc_degree_body)


def _sc_degree_body(dst_hbm, out_hbm, dstbuf, hist, rowidx, acc):
    c = lax.axis_index("c")
    s = lax.axis_index("s")
    wid = s * NC + c

    def _zrow(i, _):
        def _zcol(k, _):
            hist[i, pl.ds(k * 16, 16)] = jnp.zeros((16,), jnp.float32)
            return 0
        return lax.fori_loop(0, HALF // 16, _zcol, 0)

    lax.fori_loop(0, DEGROWS, _zrow, 0)

    def _ridx(k, _):
        rowidx[pl.ds(k * 16, 16)] = lax.iota(jnp.int32, 16) + k * 16
        return 0

    lax.fori_loop(0, DEGROWS // 16, _ridx, 0)

    pltpu.sync_copy(hist.at[pl.ds(s * 5, 5)], acc.at[pl.ds(s * 5, 5)])
    plsc.subcore_barrier()

    pltpu.sync_copy(dst_hbm.at[wid], dstbuf.at[pl.ds(0, EPW)])

    ones = jnp.ones((16,), jnp.float32)

    def _acc(i, _):
        d = dstbuf[pl.ds(i * 16, 16)]
        plsc.addupdate_scatter(hist, [d >> 7, d & 127], ones)
        return 0

    nfull = EPW // 16
    lax.fori_loop(0, nfull, _acc, 0)
    rem = EPW - nfull * 16
    if rem:
        d = dstbuf[pl.ds(nfull * 16, 16)]
        m = lax.iota(jnp.int32, 16) < rem
        plsc.addupdate_scatter(hist, [d >> 7, d & 127], ones, mask=m)

    pltpu.sync_copy(hist, acc.at[rowidx], add=True)
    plsc.subcore_barrier()
    pltpu.sync_copy(acc.at[pl.ds(s * 5, 5)], out_hbm.at[c, pl.ds(s * 5, 5)])


@functools.lru_cache(maxsize=None)
def _build_sc_scatter():
    mesh = plsc.VectorSubcoreMesh(core_axis_name="c", subcore_axis_name="s",
                                  num_cores=NC, num_subcores=NS)
    return functools.partial(
        pl.kernel,
        out_type=jax.ShapeDtypeStruct((NC, NPAD, HALF), jnp.float32),
        mesh=mesh,
        scratch_types=[
            pltpu.VMEM((IDP, 2, EW), jnp.int32),
            pltpu.VMEM((NBUF, EW, HALF), jnp.float32),
            pltpu.VMEM((32, HALF), jnp.float32),
            pltpu.VMEM_SHARED((NPAD, HALF), jnp.float32),
            [pltpu.SemaphoreType.DMA] * NBUF,
            [pltpu.SemaphoreType.DMA] * NBUF,
            [pltpu.SemaphoreType.DMA] * IDP,
        ],
        compiler_params=pltpu.CompilerParams(use_tc_tiling_on_sc=False, needs_layout_passes=False),
    )(_sc_scatter_body)


def _sc_scatter_body(hs_hbm, eidx_hbm, zero_hbm, out_hbm,
                     idxring, rowbuf, zbuf, acc, gsems, ssems, esems):
    c = lax.axis_index("c")
    s = lax.axis_index("s")

    pltpu.sync_copy(zero_hbm, zbuf)
    base = s * ROWS_PER_TILE
    for k in range(ROWS_PER_TILE // 32):
        pltpu.sync_copy(zbuf, acc.at[pl.ds(base + k * 32, 32)])

    hsrc = hs_hbm.at[c]
    echunk = eidx_hbm.at[s]

    def _load_idx(j, p):
        pltpu.async_copy(echunk.at[j], idxring.at[p], esems[p])

    def _wait_idx(p):
        pltpu.make_async_copy(echunk.at[0], idxring.at[p], esems[p]).wait()

    def _gather(j_p, b):
        pltpu.async_copy(hsrc.at[idxring.at[j_p, 0]], rowbuf.at[b], gsems[b])

    def _wait_gather(b):
        pltpu.make_async_copy(hsrc.at[idxring.at[0, 0]], rowbuf.at[b],
                              gsems[b]).wait()

    def _scatter(j_p, b):
        pltpu.async_copy(rowbuf.at[b], acc.at[idxring.at[j_p, 1]], ssems[b],
                         add=True)

    def _wait_scatter(b):
        pltpu.make_async_copy(rowbuf.at[b], acc.at[idxring.at[0, 1]],
                              ssems[b]).wait()

    plsc.subcore_barrier()

    for p in range(IDP):
        _load_idx(p, p)
    _wait_idx(0)
    _gather(0, 0)
    for j in (0, 1):
        _wait_gather(j % NBUF)
        _scatter(j % IDP, j % NBUF)
        _wait_idx((j + 1) % IDP)
        _gather((j + 1) % IDP, (j + 1) % NBUF)

    def _round(gg, _):
        j0 = 2 + IDP * gg
        for q in range(IDP):
            j = j0 + q
            b = (2 + q) % NBUF
            p = (2 + q) % IDP
            b1 = (3 + q) % NBUF
            p1 = (3 + q) % IDP
            _wait_gather(b)
            _scatter(p, b)
            _wait_scatter(b1)
            _load_idx(j + 4, q % IDP)
            _wait_idx(p1)
            _gather(p1, b1)
        return 0

    lax.fori_loop(0, (EB - 2 - 8) // IDP, _round, 0)

    for j in range(EB - 8, EB):
        _wait_gather(j % NBUF)
        _scatter(j % IDP, j % NBUF)
        if j < EB - 1:
            _wait_scatter((j + 1) % NBUF)
            if j + 4 <= EB - 1:
                _load_idx(j + 4, (j + 4) % IDP)
            _wait_idx((j + 1) % IDP)
            _gather((j + 1) % IDP, (j + 1) % NBUF)
    for j in range(EB - 3, EB):
        _wait_scatter(j % NBUF)

    plsc.subcore_barrier()
    pltpu.sync_copy(acc.at[pl.ds(base, ROWS_PER_TILE)],
                    out_hbm.at[c, pl.ds(base, ROWS_PER_TILE)])


def _tc1_body(x_ref, w_ref, d0_ref, d1_ref, hs_ref, h_ref, dinv_ref):
    deg = d0_ref[...] + d1_ref[...] + 1.0
    dinv = lax.rsqrt(jnp.maximum(deg, 1.0))
    h = jnp.dot(x_ref[...], w_ref[...], preferred_element_type=jnp.float32)
    hs = h * dinv
    hs_ref[0, :, :] = hs[:, :HALF]
    hs_ref[1, :, :] = hs[:, HALF:]
    h_ref[...] = h
    dinv_ref[...] = dinv


def _tc2_body(s_ref, h1_ref, dinv_ref, b1_ref, w2_ref, hs2_ref, h2_ref):
    dinv = dinv_ref[...]
    sfull = jnp.concatenate([s_ref[0], s_ref[1]], axis=1)
    h2 = jnp.tanh(dinv * sfull + (dinv * dinv) * h1_ref[...] + b1_ref[...])
    hh = jnp.dot(h2, w2_ref[...], preferred_element_type=jnp.float32)
    hss = hh * dinv
    hs2_ref[0, :, :] = hss[:, :HALF]
    hs2_ref[1, :, :] = hss[:, HALF:]
    h2_ref[...] = hh


def _tc3_body(s_ref, h2_ref, dinv_ref, b2_ref, ids_ref, out_ref,
              accs_ref, accc_ref):
    i = pl.program_id(0)

    @pl.when(i == 0)
    def _init():
        accs_ref[...] = jnp.zeros_like(accs_ref)
        accc_ref[...] = jnp.zeros_like(accc_ref)

    dinv = dinv_ref[...]
    sfull = jnp.concatenate([s_ref[0], s_ref[1]], axis=1)
    h = jnp.tanh(dinv * sfull + (dinv * dinv) * h2_ref[...] + b2_ref[...])
    nrm = jnp.maximum(jnp.sqrt(jnp.sum(h * h, axis=1, keepdims=True)), EPSN)
    hp = jnp.where(nrm > MAXNORM, h * (MAXNORM / nrm), h)
    n2 = jnp.maximum(jnp.sqrt(jnp.sum(hp * hp, axis=1, keepdims=True)), EPSN)
    z = jnp.minimum(n2, 1.0 - 1e-7)
    atz = 0.5 * jnp.log((1.0 + z) / (1.0 - z))
    u = (atz / n2) * hp
    ids = ids_ref[...]
    segi = lax.broadcasted_iota(jnp.int32, (NSEG, RB), 0)
    oh = (segi == ids).astype(jnp.float32)
    accs_ref[...] += lax.dot_general(
        oh, u, (((1,), (0,)), ((), ())), preferred_element_type=jnp.float32)
    accc_ref[...] += lax.dot_general(
        oh, jnp.ones((RB, 8), jnp.float32), (((1,), (0,)), ((), ())),
        preferred_element_type=jnp.float32)

    @pl.when(i == GRID - 1)
    def _final():
        cnt = accc_ref[...][:, :1]
        mean = accs_ref[...] / jnp.maximum(cnt, 1.0)
        nm = jnp.maximum(jnp.sqrt(jnp.sum(mean * mean, axis=1, keepdims=True)), EPSN)
        em = (jnp.tanh(nm) / nm) * mean
        ne = jnp.maximum(jnp.sqrt(jnp.sum(em * em, axis=1, keepdims=True)), EPSN)
        out_ref[...] = jnp.where(ne > MAXNORM, em * (MAXNORM / ne), em)


_f32 = jnp.float32


def _tc1(xp, W1, d0, d1):
    return pl.pallas_call(
        _tc1_body,
        grid=(GRID,),
        in_specs=[
            pl.BlockSpec((RB, D), lambda i: (i, 0)),
            pl.BlockSpec((D, D), lambda i: (0, 0)),
            pl.BlockSpec((RB, 1), lambda i: (i, 0)),
            pl.BlockSpec((RB, 1), lambda i: (i, 0)),
        ],
        out_specs=[
            pl.BlockSpec((NC, RB, HALF), lambda i: (0, i, 0)),
            pl.BlockSpec((RB, D), lambda i: (i, 0)),
            pl.BlockSpec((RB, 1), lambda i: (i, 0)),
        ],
        out_shape=[
            jax.ShapeDtypeStruct((NC, NPAD, HALF), _f32),
            jax.ShapeDtypeStruct((NPAD, D), _f32),
            jax.ShapeDtypeStruct((NPAD, 1), _f32),
        ],
    )(xp, W1, d0, d1)


def _tc2(s1, h1, dinv, b1, W2):
    return pl.pallas_call(
        _tc2_body,
        grid=(GRID,),
        in_specs=[
            pl.BlockSpec((NC, RB, HALF), lambda i: (0, i, 0)),
            pl.BlockSpec((RB, D), lambda i: (i, 0)),
            pl.BlockSpec((RB, 1), lambda i: (i, 0)),
            pl.BlockSpec((1, D), lambda i: (0, 0)),
            pl.BlockSpec((D, D), lambda i: (0, 0)),
        ],
        out_specs=[
            pl.BlockSpec((NC, RB, HALF), lambda i: (0, i, 0)),
            pl.BlockSpec((RB, D), lambda i: (i, 0)),
        ],
        out_shape=[
            jax.ShapeDtypeStruct((NC, NPAD, HALF), _f32),
            jax.ShapeDtypeStruct((NPAD, D), _f32),
        ],
    )(s1, h1, dinv, b1, W2)


def _tc3(s2, h2, dinv, b2, ids_row):
    return pl.pallas_call(
        _tc3_body,
        grid=(GRID,),
        in_specs=[
            pl.BlockSpec((NC, RB, HALF), lambda i: (0, i, 0)),
            pl.BlockSpec((RB, D), lambda i: (i, 0)),
            pl.BlockSpec((RB, 1), lambda i: (i, 0)),
            pl.BlockSpec((1, D), lambda i: (0, 0)),
            pl.BlockSpec((1, RB), lambda i: (0, i)),
        ],
        out_specs=pl.BlockSpec((NSEG, D), lambda i: (0, 0)),
        out_shape=jax.ShapeDtypeStruct((NSEG, D), _f32),
        scratch_shapes=[
            pltpu.VMEM((NSEG, D), _f32),
            pltpu.VMEM((NSEG, 8), _f32),
        ],
    )(s2, h2, dinv, b2, ids_row)


def kernel(x, edge_index, comment_ids, W1, b1, W2, b2):
    xp = jnp.pad(x, ((0, NPAD - N), (0, 0)))
    src_sc = edge_index[0].reshape(NS, EB, EW)
    dst_sc = edge_index[1].reshape(NS, EB, EW)
    eidx = jnp.stack([src_sc, dst_sc], axis=2)
    dst_deg = edge_index[1].reshape(NC * NS, EPW)
    ids_row = jnp.pad(comment_ids.astype(jnp.int32), (0, NPAD - N),
                      constant_values=-1).reshape(1, NPAD)
    zeros_hbm = jnp.zeros((32, HALF), _f32)

    degout = _build_sc_degree()(dst_deg)
    d0 = degout[0].reshape(NPAD, 1)
    d1 = degout[1].reshape(NPAD, 1)

    hs1, h1, dinv = _tc1(xp, W1, d0, d1)
    s1 = _build_sc_scatter()(hs1, eidx, zeros_hbm)
    hs2, h2 = _tc2(s1, h1, dinv, b1.reshape(1, D), W2)
    s2 = _build_sc_scatter()(hs2, eidx, zeros_hbm)
    out = _tc3(s2, h2, dinv, b2.reshape(1, D), ids_row)
    return out.reshape(16, 32, D)
